# baseline (speedup 1.0000x reference)
import functools

import jax
import jax.numpy as jnp
from jax import lax
from jax.experimental import pallas as pl
from jax.experimental.pallas import tpu as pltpu
from jax.experimental.pallas import tpu_sc as plsc

N = 1000000
EMBED_DIM = 32
BATCH = 16384
FIELDS = 26

NC = 2
NS = 16
NW = NC * NS

TOTAL = BATCH * FIELDS
PER_W = TOTAL // NW
CHUNK = 128
NCHUNK = PER_W // CHUNK
G = 8
NGROUP = NCHUNK // G

assert PER_W * NW == TOTAL
assert CHUNK * NCHUNK == PER_W
assert G * NGROUP == NCHUNK


GROWS = G * CHUNK

VBLK = 512
NB = 1954
VPAD = NB * VBLK


def _tr_body(tt_ref, o_ref):
    o_ref[...] = jnp.concatenate(
        [tt_ref[:, 128 * j : 128 * (j + 1)].T for j in range(4)], axis=1
    )


_transpose = pl.pallas_call(
    _tr_body,
    grid=(NB,),
    in_specs=[pl.BlockSpec((32, VBLK), lambda i: (0, i))],
    out_specs=pl.BlockSpec((128, 128), lambda i: (i, 0)),
    out_shape=jax.ShapeDtypeStruct((NB * 128, 128), jnp.float32),
)


def _body(x_hbm, table_hbm, out_hbm, idx_v, rows0, rows1, gs0, gs1, ws0, ws1):
    c = lax.axis_index("c")
    s = lax.axis_index("s")
    wid = s * NC + c
    base = wid * PER_W
    pltpu.sync_copy(x_hbm.at[wid], idx_v)

    bufs = ((rows0, gs0, ws0), (rows1, gs1, ws1))

    def fire_gather(g):
        buf, gsem, _ = bufs[g % 2]
        return [
            pltpu.async_copy(
                table_hbm.at[idx_v.at[g * G + b]],
                buf.at[pl.ds(b * CHUNK, CHUNK)],
                gsem,
            )
            for b in range(G)
        ]

    gpend = {0: fire_gather(0), 1: fire_gather(1)}
    wpend = {}
    for g in range(NGROUP):
        buf, _, wsem = bufs[g % 2]
        for cp in gpend.pop(g):
            cp.wait()
        wpend[g] = pltpu.async_copy(
            buf, out_hbm.at[pl.ds(base + g * GROWS, GROWS)], wsem
        )
        if g + 2 < NGROUP:
            wpend.pop(g).wait()
            gpend[g + 2] = fire_gather(g + 2)
    for cp in wpend.values():
        cp.wait()


_mesh = plsc.VectorSubcoreMesh(
    core_axis_name="c", subcore_axis_name="s", num_cores=NC, num_subcores=NS
)

_sc_gather = pl.kernel(
    _body,
    out_type=jax.ShapeDtypeStruct((TOTAL, EMBED_DIM), jnp.float32),
    mesh=_mesh,
    scratch_types=[
        pltpu.VMEM((NCHUNK, CHUNK), jnp.int32),
        pltpu.VMEM((GROWS, EMBED_DIM), jnp.float32),
        pltpu.VMEM((GROWS, EMBED_DIM), jnp.float32),
        pltpu.SemaphoreType.DMA,
        pltpu.SemaphoreType.DMA,
        pltpu.SemaphoreType.DMA,
        pltpu.SemaphoreType.DMA,
    ],
    compiler_params=pltpu.CompilerParams(use_tc_tiling_on_sc=False),
)


@jax.jit
def kernel(x, table):
    tbl_rm = _transpose(table.T).reshape(VPAD, EMBED_DIM)
    xi = x.astype(jnp.int32)
    idx = ((xi & ~511) | ((xi & 127) << 2) | ((xi >> 7) & 3)).reshape(
        NW, NCHUNK, CHUNK
    )
    out = _sc_gather(idx, tbl_rm)
    return out.reshape(BATCH, FIELDS, EMBED_DIM)


# speedup vs baseline: 2.8833x; 2.8833x over previous
import functools

import jax
import jax.numpy as jnp
from jax import lax
from jax.experimental import pallas as pl
from jax.experimental.pallas import tpu as pltpu
from jax.experimental.pallas import tpu_sc as plsc

N = 1000000
EMBED_DIM = 32
BATCH = 16384
FIELDS = 26

NC = 2
NS = 16
NW = NC * NS

TOTAL = BATCH * FIELDS
PER_W = TOTAL // NW
CHUNK = 128
NCHUNK = PER_W // CHUNK
G = 8
NGROUP = NCHUNK // G

assert PER_W * NW == TOTAL
assert CHUNK * NCHUNK == PER_W
assert G * NGROUP == NCHUNK


GROWS = G * CHUNK

VBLK = 4096
NB = 245
VPAD = NB * VBLK


def _tr_body(tt_ref, o_ref):
    for g in range(VBLK // 512):
        m = jnp.concatenate(
            [tt_ref[:, 512 * g + 128 * j : 512 * g + 128 * (j + 1)] for j in range(4)],
            axis=0,
        )
        o_ref[128 * g : 128 * (g + 1), :] = m.T


_transpose = pl.pallas_call(
    _tr_body,
    grid=(NB,),
    in_specs=[pl.BlockSpec((32, VBLK), lambda i: (0, i))],
    out_specs=pl.BlockSpec((VBLK // 4, 128), lambda i: (i, 0)),
    out_shape=jax.ShapeDtypeStruct((NB * VBLK // 4, 128), jnp.float32),
)


def _body(x_hbm, table_hbm, out_hbm, idx_v, rows0, rows1, gs0, gs1, ws0, ws1):
    c = lax.axis_index("c")
    s = lax.axis_index("s")
    wid = s * NC + c
    base = wid * PER_W
    pltpu.sync_copy(x_hbm.at[wid], idx_v)

    bufs = ((rows0, gs0, ws0), (rows1, gs1, ws1))

    def fire_gather(g):
        buf, gsem, _ = bufs[g % 2]
        return [
            pltpu.async_copy(
                table_hbm.at[idx_v.at[g * G + b]],
                buf.at[pl.ds(b * CHUNK, CHUNK)],
                gsem,
            )
            for b in range(G)
        ]

    gpend = {0: fire_gather(0), 1: fire_gather(1)}
    wpend = {}
    for g in range(NGROUP):
        buf, _, wsem = bufs[g % 2]
        for cp in gpend.pop(g):
            cp.wait()
        wpend[g] = pltpu.async_copy(
            buf, out_hbm.at[pl.ds(base + g * GROWS, GROWS)], wsem
        )
        if g + 2 < NGROUP:
            wpend.pop(g).wait()
            gpend[g + 2] = fire_gather(g + 2)
    for cp in wpend.values():
        cp.wait()


_mesh = plsc.VectorSubcoreMesh(
    core_axis_name="c", subcore_axis_name="s", num_cores=NC, num_subcores=NS
)

_sc_gather = pl.kernel(
    _body,
    out_type=jax.ShapeDtypeStruct((TOTAL, EMBED_DIM), jnp.float32),
    mesh=_mesh,
    scratch_types=[
        pltpu.VMEM((NCHUNK, CHUNK), jnp.int32),
        pltpu.VMEM((GROWS, EMBED_DIM), jnp.float32),
        pltpu.VMEM((GROWS, EMBED_DIM), jnp.float32),
        pltpu.SemaphoreType.DMA,
        pltpu.SemaphoreType.DMA,
        pltpu.SemaphoreType.DMA,
        pltpu.SemaphoreType.DMA,
    ],
    compiler_params=pltpu.CompilerParams(use_tc_tiling_on_sc=False),
)


@jax.jit
def kernel(x, table):
    tbl_rm = _transpose(table.T).reshape(VPAD, EMBED_DIM)
    xi = x.astype(jnp.int32)
    idx = ((xi & ~511) | ((xi & 127) << 2) | ((xi >> 7) & 3)).reshape(
        NW, NCHUNK, CHUNK
    )
    out = _sc_gather(idx, tbl_rm)
    return out.reshape(BATCH, FIELDS, EMBED_DIM)


# speedup vs baseline: 3.2487x; 1.1267x over previous
import functools

import jax
import jax.numpy as jnp
from jax import lax
from jax.experimental import pallas as pl
from jax.experimental.pallas import tpu as pltpu
from jax.experimental.pallas import tpu_sc as plsc

N = 1000000
EMBED_DIM = 32
BATCH = 16384
FIELDS = 26

NC = 2
NS = 16
NW = NC * NS

TOTAL = BATCH * FIELDS
PER_W = TOTAL // NW
CHUNK = 128
NCHUNK = PER_W // CHUNK
G = 8
NGROUP = NCHUNK // G

assert PER_W * NW == TOTAL
assert CHUNK * NCHUNK == PER_W
assert G * NGROUP == NCHUNK


GROWS = G * CHUNK

VBLK = 8192
NB = 123
VPAD = NB * VBLK


def _tr_body(tt_ref, o_ref):
    for g in range(VBLK // 512):
        m = jnp.concatenate(
            [tt_ref[:, 512 * g + 128 * j : 512 * g + 128 * (j + 1)] for j in range(4)],
            axis=0,
        )
        o_ref[128 * g : 128 * (g + 1), :] = m.T


_transpose = pl.pallas_call(
    _tr_body,
    grid=(NB,),
    in_specs=[pl.BlockSpec((32, VBLK), lambda i: (0, i))],
    out_specs=pl.BlockSpec((VBLK // 4, 128), lambda i: (i, 0)),
    out_shape=jax.ShapeDtypeStruct((NB * VBLK // 4, 128), jnp.float32),
)


def _body(x_hbm, table_hbm, out_hbm, idx_v, rows0, rows1, gs0, gs1, ws0, ws1):
    c = lax.axis_index("c")
    s = lax.axis_index("s")
    wid = s * NC + c
    base = wid * PER_W
    pltpu.sync_copy(x_hbm.at[wid], idx_v)

    bufs = ((rows0, gs0, ws0), (rows1, gs1, ws1))

    def fire_gather(g):
        buf, gsem, _ = bufs[g % 2]
        return [
            pltpu.async_copy(
                table_hbm.at[idx_v.at[g * G + b]],
                buf.at[pl.ds(b * CHUNK, CHUNK)],
                gsem,
            )
            for b in range(G)
        ]

    gpend = {0: fire_gather(0), 1: fire_gather(1)}
    wpend = {}
    for g in range(NGROUP):
        buf, _, wsem = bufs[g % 2]
        for cp in gpend.pop(g):
            cp.wait()
        wpend[g] = pltpu.async_copy(
            buf, out_hbm.at[pl.ds(base + g * GROWS, GROWS)], wsem
        )
        if g + 2 < NGROUP:
            wpend.pop(g).wait()
            gpend[g + 2] = fire_gather(g + 2)
    for cp in wpend.values():
        cp.wait()


_mesh = plsc.VectorSubcoreMesh(
    core_axis_name="c", subcore_axis_name="s", num_cores=NC, num_subcores=NS
)

_sc_gather = pl.kernel(
    _body,
    out_type=jax.ShapeDtypeStruct((TOTAL, EMBED_DIM), jnp.float32),
    mesh=_mesh,
    scratch_types=[
        pltpu.VMEM((NCHUNK, CHUNK), jnp.int32),
        pltpu.VMEM((GROWS, EMBED_DIM), jnp.float32),
        pltpu.VMEM((GROWS, EMBED_DIM), jnp.float32),
        pltpu.SemaphoreType.DMA,
        pltpu.SemaphoreType.DMA,
        pltpu.SemaphoreType.DMA,
        pltpu.SemaphoreType.DMA,
    ],
    compiler_params=pltpu.CompilerParams(use_tc_tiling_on_sc=False),
)


@jax.jit
def kernel(x, table):
    tbl_rm = _transpose(table.T).reshape(VPAD, EMBED_DIM)
    xi = x.astype(jnp.int32)
    idx = ((xi & ~511) | ((xi & 127) << 2) | ((xi >> 7) & 3)).reshape(
        NW, NCHUNK, CHUNK
    )
    out = _sc_gather(idx, tbl_rm)
    return out.reshape(BATCH, FIELDS, EMBED_DIM)


# speedup vs baseline: 3.5435x; 1.0907x over previous
import functools

import jax
import jax.numpy as jnp
from jax import lax
from jax.experimental import pallas as pl
from jax.experimental.pallas import tpu as pltpu
from jax.experimental.pallas import tpu_sc as plsc

N = 1000000
EMBED_DIM = 32
BATCH = 16384
FIELDS = 26

NC = 2
NS = 16
NW = NC * NS

TOTAL = BATCH * FIELDS
PER_W = TOTAL // NW
CHUNK = 128
NCHUNK = PER_W // CHUNK
G = 8
NGROUP = NCHUNK // G

assert PER_W * NW == TOTAL
assert CHUNK * NCHUNK == PER_W
assert G * NGROUP == NCHUNK


GROWS = G * CHUNK

VBLK = 16384
NB = 62
VPAD = NB * VBLK


def _tr_body(tt_ref, o_ref):
    for g in range(VBLK // 512):
        m = jnp.concatenate(
            [tt_ref[:, 512 * g + 128 * j : 512 * g + 128 * (j + 1)] for j in range(4)],
            axis=0,
        )
        o_ref[128 * g : 128 * (g + 1), :] = m.T


_transpose = pl.pallas_call(
    _tr_body,
    grid=(NB,),
    in_specs=[pl.BlockSpec((32, VBLK), lambda i: (0, i))],
    out_specs=pl.BlockSpec((VBLK // 4, 128), lambda i: (i, 0)),
    out_shape=jax.ShapeDtypeStruct((NB * VBLK // 4, 128), jnp.float32),
)


def _body(x_hbm, table_hbm, out_hbm, idx_v, rows0, rows1, gs0, gs1, ws0, ws1):
    c = lax.axis_index("c")
    s = lax.axis_index("s")
    wid = s * NC + c
    base = wid * PER_W
    pltpu.sync_copy(x_hbm.at[wid], idx_v)

    bufs = ((rows0, gs0, ws0), (rows1, gs1, ws1))

    def fire_gather(g):
        buf, gsem, _ = bufs[g % 2]
        return [
            pltpu.async_copy(
                table_hbm.at[idx_v.at[g * G + b]],
                buf.at[pl.ds(b * CHUNK, CHUNK)],
                gsem,
            )
            for b in range(G)
        ]

    gpend = {0: fire_gather(0), 1: fire_gather(1)}
    wpend = {}
    for g in range(NGROUP):
        buf, _, wsem = bufs[g % 2]
        for cp in gpend.pop(g):
            cp.wait()
        wpend[g] = pltpu.async_copy(
            buf, out_hbm.at[pl.ds(base + g * GROWS, GROWS)], wsem
        )
        if g + 2 < NGROUP:
            wpend.pop(g).wait()
            gpend[g + 2] = fire_gather(g + 2)
    for cp in wpend.values():
        cp.wait()


_mesh = plsc.VectorSubcoreMesh(
    core_axis_name="c", subcore_axis_name="s", num_cores=NC, num_subcores=NS
)

_sc_gather = pl.kernel(
    _body,
    out_type=jax.ShapeDtypeStruct((TOTAL, EMBED_DIM), jnp.float32),
    mesh=_mesh,
    scratch_types=[
        pltpu.VMEM((NCHUNK, CHUNK), jnp.int32),
        pltpu.VMEM((GROWS, EMBED_DIM), jnp.float32),
        pltpu.VMEM((GROWS, EMBED_DIM), jnp.float32),
        pltpu.SemaphoreType.DMA,
        pltpu.SemaphoreType.DMA,
        pltpu.SemaphoreType.DMA,
        pltpu.SemaphoreType.DMA,
    ],
    compiler_params=pltpu.CompilerParams(use_tc_tiling_on_sc=False),
)


@jax.jit
def kernel(x, table):
    tbl_rm = _transpose(table.T).reshape(VPAD, EMBED_DIM)
    xi = x.astype(jnp.int32)
    idx = ((xi & ~511) | ((xi & 127) << 2) | ((xi >> 7) & 3)).reshape(
        NW, NCHUNK, CHUNK
    )
    out = _sc_gather(idx, tbl_rm)
    return out.reshape(BATCH, FIELDS, EMBED_DIM)


# speedup vs baseline: 3.6734x; 1.0366x over previous
import functools

import jax
import jax.numpy as jnp
from jax import lax
from jax.experimental import pallas as pl
from jax.experimental.pallas import tpu as pltpu
from jax.experimental.pallas import tpu_sc as plsc

N = 1000000
EMBED_DIM = 32
BATCH = 16384
FIELDS = 26

NC = 2
NS = 16
NW = NC * NS

TOTAL = BATCH * FIELDS
PER_W = TOTAL // NW
CHUNK = 128
NCHUNK = PER_W // CHUNK
G = 8
NGROUP = NCHUNK // G

assert PER_W * NW == TOTAL
assert CHUNK * NCHUNK == PER_W
assert G * NGROUP == NCHUNK


GROWS = G * CHUNK

VBLK = 32768
NB = 31
VPAD = NB * VBLK


def _tr_body(tt_ref, o_ref):
    for g in range(VBLK // 512):
        m = jnp.concatenate(
            [tt_ref[:, 512 * g + 128 * j : 512 * g + 128 * (j + 1)] for j in range(4)],
            axis=0,
        )
        o_ref[128 * g : 128 * (g + 1), :] = m.T


_transpose = pl.pallas_call(
    _tr_body,
    grid=(NB,),
    in_specs=[pl.BlockSpec((32, VBLK), lambda i: (0, i))],
    out_specs=pl.BlockSpec((VBLK // 4, 128), lambda i: (i, 0)),
    out_shape=jax.ShapeDtypeStruct((NB * VBLK // 4, 128), jnp.float32),
)


def _body(x_hbm, table_hbm, out_hbm, idx_v, rows0, rows1, gs0, gs1, ws0, ws1):
    c = lax.axis_index("c")
    s = lax.axis_index("s")
    wid = s * NC + c
    base = wid * PER_W
    pltpu.sync_copy(x_hbm.at[wid], idx_v)

    bufs = ((rows0, gs0, ws0), (rows1, gs1, ws1))

    def fire_gather(g):
        buf, gsem, _ = bufs[g % 2]
        return [
            pltpu.async_copy(
                table_hbm.at[idx_v.at[g * G + b]],
                buf.at[pl.ds(b * CHUNK, CHUNK)],
                gsem,
            )
            for b in range(G)
        ]

    gpend = {0: fire_gather(0), 1: fire_gather(1)}
    wpend = {}
    for g in range(NGROUP):
        buf, _, wsem = bufs[g % 2]
        for cp in gpend.pop(g):
            cp.wait()
        wpend[g] = pltpu.async_copy(
            buf, out_hbm.at[pl.ds(base + g * GROWS, GROWS)], wsem
        )
        if g + 2 < NGROUP:
            wpend.pop(g).wait()
            gpend[g + 2] = fire_gather(g + 2)
    for cp in wpend.values():
        cp.wait()


_mesh = plsc.VectorSubcoreMesh(
    core_axis_name="c", subcore_axis_name="s", num_cores=NC, num_subcores=NS
)

_sc_gather = pl.kernel(
    _body,
    out_type=jax.ShapeDtypeStruct((TOTAL, EMBED_DIM), jnp.float32),
    mesh=_mesh,
    scratch_types=[
        pltpu.VMEM((NCHUNK, CHUNK), jnp.int32),
        pltpu.VMEM((GROWS, EMBED_DIM), jnp.float32),
        pltpu.VMEM((GROWS, EMBED_DIM), jnp.float32),
        pltpu.SemaphoreType.DMA,
        pltpu.SemaphoreType.DMA,
        pltpu.SemaphoreType.DMA,
        pltpu.SemaphoreType.DMA,
    ],
    compiler_params=pltpu.CompilerParams(use_tc_tiling_on_sc=False),
)


@jax.jit
def kernel(x, table):
    tbl_rm = _transpose(table.T).reshape(VPAD, EMBED_DIM)
    xi = x.astype(jnp.int32)
    idx = ((xi & ~511) | ((xi & 127) << 2) | ((xi >> 7) & 3)).reshape(
        NW, NCHUNK, CHUNK
    )
    out = _sc_gather(idx, tbl_rm)
    return out.reshape(BATCH, FIELDS, EMBED_DIM)
